# baseline (speedup 1.0000x reference)
import functools
import math

import jax
import jax.numpy as jnp
from jax import lax
from jax.experimental import pallas as pl
from jax.experimental.pallas import tpu as pltpu
from jax.experimental.pallas import tpu_sc as plsc

B, T, D = 128, 256, 1024
V = 100000
VPAD = 100352
MAXK = 64

NW = 32
RPW = B // NW
CH = 12544
NCH = VPAD // CH
NGRP = CH // 128
RES = 320
NRV = RES // 16
TRIG = 192
NEG_INIT = -3.4e38


def _pool_router_body(hidden_ref, w_ref, b_ref, out_ref, acc_ref):
    t = pl.program_id(0)
    nt = pl.num_programs(0)

    @pl.when(t == 0)
    def _init():
        acc_ref[...] = jnp.zeros_like(acc_ref)

    acc_ref[...] += jnp.sum(hidden_ref[...], axis=1)

    @pl.when(t == nt - 1)
    def _fin():
        pooled = acc_ref[...] * (1.0 / T)
        out_ref[...] = (
            jnp.dot(pooled, w_ref[...], preferred_element_type=jnp.float32)
            + b_ref[...][None, :]
        )


def _pool_router(hidden, w, b):
    tb = 32
    return pl.pallas_call(
        _pool_router_body,
        grid=(T // tb,),
        in_specs=[
            pl.BlockSpec((B, tb, D), lambda t: (0, t, 0)),
            pl.BlockSpec((D, D), lambda t: (0, 0)),
            pl.BlockSpec((D,), lambda t: (0,)),
        ],
        out_specs=pl.BlockSpec((B, D), lambda t: (0, 0)),
        out_shape=jax.ShapeDtypeStruct((B, D), jnp.float32),
        scratch_shapes=[pltpu.VMEM((B, D), jnp.float32)],
    )(hidden, w, b)


def _scores_body(q_ref, k_ref, out_ref, *, vb):
    j = pl.program_id(0)
    s = lax.dot_general(
        q_ref[...], k_ref[...],
        dimension_numbers=(((1,), (1,)), ((), ())),
        preferred_element_type=jnp.float32,
    ) * (1.0 / math.sqrt(D))
    col = j * vb + lax.broadcasted_iota(jnp.int32, (B, vb), 1)
    out_ref[...] = jnp.where(col < V, s, -3.0e38)


def _scores(query, keys_tbl):
    vb = 2048
    return pl.pallas_call(
        functools.partial(_scores_body, vb=vb),
        grid=(VPAD // vb,),
        in_specs=[
            pl.BlockSpec((B, D), lambda j: (0, 0)),
            pl.BlockSpec((vb, D), lambda j: (j, 0)),
        ],
        out_specs=pl.BlockSpec((B, vb), lambda j: (0, j)),
        out_shape=jax.ShapeDtypeStruct((B, VPAD), jnp.float32),
    )(query, keys_tbl)


def _f2k(v):
    u = lax.bitcast_convert_type(v, jnp.uint32)
    return jnp.where(u >= jnp.uint32(0x80000000), ~u,
                     u | jnp.uint32(0x80000000))


def _iota16():
    return lax.iota(jnp.int32, 16)


def _rebuild(cand_v, cand_i, key_b, tie_v, tie_i, cnt):
    iot = _iota16()
    for j in range(NRV):
        vj = cand_v[pl.ds(j * 16, 16)]
        kj = _f2k(vj)
        lane = j * 16 + iot
        key_b[pl.ds(j * 16, 16)] = jnp.where(lane < cnt, kj, jnp.uint32(0))

    def bit_body(i, tk):
        cand_t = tk | (jnp.uint32(1) << (jnp.uint32(31) - i.astype(jnp.uint32)))
        tot = jnp.zeros((16,), jnp.int32)
        for j in range(NRV):
            kj = key_b[pl.ds(j * 16, 16)]
            tot = tot + (kj >= cand_t).astype(jnp.int32)
        return jnp.where(jnp.sum(tot) >= MAXK, cand_t, tk)

    tkey = lax.fori_loop(0, 32, bit_body, jnp.uint32(0))

    n_gt = jnp.int32(0)
    n_eq = jnp.int32(0)
    for j in range(NRV):
        kj = key_b[pl.ds(j * 16, 16)]
        vj = cand_v[pl.ds(j * 16, 16)]
        ij = cand_i[pl.ds(j * 16, 16)]
        mgt = kj > tkey
        meq = kj == tkey
        rgt = jnp.cumsum(mgt.astype(jnp.int32))
        req = jnp.cumsum(meq.astype(jnp.int32))
        pgt = n_gt + rgt - 1
        peq = n_eq + req - 1
        plsc.store_scatter(cand_v, [pgt], vj, mask=mgt)
        plsc.store_scatter(cand_i, [pgt], ij, mask=mgt)
        mke = meq & (peq < MAXK)
        plsc.store_scatter(tie_v, [peq], vj, mask=mke)
        plsc.store_scatter(tie_i, [peq], ij, mask=mke)
        n_gt = n_gt + rgt[15]
        n_eq = n_eq + req[15]

    need = MAXK - n_gt
    for jj in range(MAXK // 16):
        lidx = jj * 16 + iot
        mk = lidx < need
        tv = tie_v[pl.ds(jj * 16, 16)]
        ti = tie_i[pl.ds(jj * 16, 16)]
        plsc.store_scatter(cand_v, [n_gt + lidx], tv, mask=mk)
        plsc.store_scatter(cand_i, [n_gt + lidx], ti, mask=mk)

    tvv = jnp.full((16,), tkey, jnp.uint32)
    uv = jnp.where(tvv < jnp.uint32(0x80000000), ~tvv,
                   tvv ^ jnp.uint32(0x80000000))
    tf = lax.bitcast_convert_type(uv, jnp.float32)
    return tf, jnp.int32(MAXK)


def _topk_body(scores, out_v, out_i,
               buf0, buf1, cand_v, cand_i, key_b, tie_v, tie_i, key_s,
               ov, oi, sem0, sem1):
    wid = lax.axis_index("s") * 2 + lax.axis_index("c")
    iot = _iota16()

    def row_body(r, _carry):
        row = wid * RPW + r

        def src(c):
            return scores.at[row, pl.ds(c * CH, CH)]

        def process(bref, cbase, tv, cnt):
            def grp(g, carry2):
                tv, cnt = carry2
                off = g * 128
                vs = [bref[pl.ds(off + 16 * j, 16)] for j in range(8)]
                m = vs[0]
                for j in range(1, 8):
                    m = jnp.maximum(m, vs[j])
                nhot = plsc.all_reduce_population_count(m > tv)

                def slow(tv_in, cnt_in):
                    cc = cnt_in
                    for j in range(8):
                        vj = bref[pl.ds(off + 16 * j, 16)]
                        mj = vj > tv_in
                        cj = plsc.all_reduce_population_count(mj)

                        def app(ci, vj=vj, mj=mj, j=j):
                            rk = jnp.cumsum(mj.astype(jnp.int32))
                            pos = ci + rk - 1
                            gi = cbase + off + 16 * j + iot
                            plsc.store_scatter(cand_v, [pos], vj, mask=mj)
                            plsc.store_scatter(cand_i, [pos], gi, mask=mj)
                            return ci + rk[15]

                        cc = lax.cond(cj[0] > 0, app, lambda ci: ci, cc)
                    return lax.cond(
                        cc > TRIG,
                        lambda a, b: _rebuild(cand_v, cand_i, key_b, tie_v,
                                              tie_i, b),
                        lambda a, b: (a, b), tv_in, cc)

                tv, cnt = lax.cond(nhot[0] > 0, slow,
                                   lambda a, b: (a, b), tv, cnt)
                return tv, cnt

            return lax.fori_loop(0, NGRP, grp, (tv, cnt))

        pltpu.async_copy(src(0), buf0, sem0)

        def pair_body(p, carry):
            tv, cnt = carry
            c0 = 2 * p
            pltpu.make_async_copy(src(c0), buf0, sem0).wait()
            pltpu.async_copy(src(c0 + 1), buf1, sem1)
            tv, cnt = process(buf0, c0 * CH, tv, cnt)
            pltpu.make_async_copy(src(c0 + 1), buf1, sem1).wait()

            @pl.when(p < NCH // 2 - 1)
            def _pref():
                pltpu.async_copy(src(c0 + 2), buf0, sem0)

            tv, cnt = process(buf1, (c0 + 1) * CH, tv, cnt)
            return tv, cnt

        tv, cnt = lax.fori_loop(
            0, NCH // 2, pair_body,
            (jnp.full((16,), jnp.float32(NEG_INIT)), jnp.int32(0)))

        tv, cnt = _rebuild(cand_v, cand_i, key_b, tie_v, tie_i, cnt)

        for j in range(MAXK // 16):
            kj = _f2k(cand_v[pl.ds(j * 16, 16)])
            key_s[pl.ds(j * 16, 16)] = lax.bitcast_convert_type(
                kj ^ jnp.uint32(0x80000000), jnp.int32)

        lane0 = iot == 0

        def sort_step(s, _c):
            ks = [key_s[pl.ds(16 * j, 16)] for j in range(4)]
            mx = jnp.maximum(jnp.maximum(ks[0], ks[1]),
                             jnp.maximum(ks[2], ks[3]))
            mxs = jnp.max(mx)
            posv = jnp.full((16,), jnp.int32(127))
            for j in range(4):
                posv = jnp.minimum(
                    posv, jnp.where(ks[j] == mxs, 16 * j + iot, 127))
            p = jnp.min(posv)
            pv = jnp.full((16,), p)
            val = plsc.load_gather(cand_v, [pv])
            idx = plsc.load_gather(cand_i, [pv])
            sv = jnp.full((16,), s)
            plsc.store_scatter(ov, [sv], val, mask=lane0)
            plsc.store_scatter(oi, [sv], idx, mask=lane0)
            plsc.store_scatter(key_s, [pv],
                               jnp.full((16,), jnp.int32(-2147483648)),
                               mask=lane0)
            return 0

        lax.fori_loop(0, MAXK, sort_step, 0)
        pltpu.sync_copy(ov, out_v.at[row])
        pltpu.sync_copy(oi, out_i.at[row])
        return 0

    lax.fori_loop(0, RPW, row_body, 0)


def _topk_sc(scores):
    mesh = plsc.VectorSubcoreMesh(core_axis_name="c", subcore_axis_name="s",
                                  num_cores=2, num_subcores=16)
    f = pl.kernel(
        _topk_body,
        compiler_params=pltpu.CompilerParams(needs_layout_passes=False),
        out_type=(jax.ShapeDtypeStruct((B, MAXK), jnp.float32),
                  jax.ShapeDtypeStruct((B, MAXK), jnp.int32)),
        mesh=mesh,
        scratch_types=[
            pltpu.VMEM((CH,), jnp.float32),
            pltpu.VMEM((CH,), jnp.float32),
            pltpu.VMEM((RES,), jnp.float32),
            pltpu.VMEM((RES,), jnp.int32),
            pltpu.VMEM((RES,), jnp.uint32),
            pltpu.VMEM((MAXK,), jnp.float32),
            pltpu.VMEM((MAXK,), jnp.int32),
            pltpu.VMEM((MAXK,), jnp.int32),
            pltpu.VMEM((MAXK,), jnp.float32),
            pltpu.VMEM((MAXK,), jnp.int32),
            pltpu.SemaphoreType.DMA,
            pltpu.SemaphoreType.DMA,
        ],
    )
    return f(scores)


def _softmax_body(v_ref, kd_ref, out_ref):
    kd = jnp.minimum(kd_ref[0], kd_ref[1])
    iota = lax.broadcasted_iota(jnp.int32, (B, MAXK), 1)
    vals = jnp.where(iota < kd, v_ref[...], -jnp.inf)
    m = jnp.max(vals, axis=1, keepdims=True)
    e = jnp.where(iota < kd, jnp.exp(vals - m), 0.0)
    out_ref[...] = e / jnp.sum(e, axis=1, keepdims=True)


def _softmax(top_vals, k_dynamic, max_k):
    kd = jnp.asarray([k_dynamic, max_k], jnp.int32).reshape(2)
    return pl.pallas_call(
        _softmax_body,
        grid=(1,),
        in_specs=[
            pl.BlockSpec((B, MAXK), lambda i: (0, 0)),
            pl.BlockSpec(memory_space=pltpu.SMEM),
        ],
        out_specs=pl.BlockSpec((B, MAXK), lambda i: (0, 0)),
        out_shape=jax.ShapeDtypeStruct((B, MAXK), jnp.float32),
    )(top_vals, kd)


def _lane_splat(x, j):
    dn = lax.GatherDimensionNumbers(offset_dims=(), collapsed_slice_dims=(0,),
                                    start_index_map=(0,))
    idx = jnp.full((16,), j, jnp.int32)
    return lax.gather(x, idx[:, None], dn, (1,),
                      mode=lax.GatherScatterMode.PROMISE_IN_BOUNDS)


def _agg_body(params, top_idx, weights, out,
              idx_v, w_v, wb, rows_v, acc, sem):
    wid = lax.axis_index("s") * 2 + lax.axis_index("c")

    def row_body(r, _carry):
        row = wid * RPW + r
        pltpu.sync_copy(top_idx.at[row], idx_v)
        pltpu.sync_copy(weights.at[row], w_v)
        cp = pltpu.async_copy(params.at[idx_v], rows_v, sem)
        for kk in range(MAXK // 16):
            wv = w_v[pl.ds(kk * 16, 16)]
            for j in range(16):
                wb[pl.ds((kk * 16 + j) * 16, 16)] = _lane_splat(wv, j)
        cp.wait()

        def d_body(dd, _c):
            a = jnp.zeros((16,), jnp.float32)
            for k in range(MAXK):
                a = a + wb[pl.ds(k * 16, 16)] * rows_v[k, pl.ds(dd * 16, 16)]
            acc[pl.ds(dd * 16, 16)] = a
            return 0

        lax.fori_loop(0, D // 16, d_body, 0)
        pltpu.sync_copy(acc, out.at[row])
        return 0

    lax.fori_loop(0, RPW, row_body, 0)


def _aggregate_sc(params_tbl, top_idx, weights):
    mesh = plsc.VectorSubcoreMesh(core_axis_name="c", subcore_axis_name="s",
                                  num_cores=2, num_subcores=16)
    f = pl.kernel(
        _agg_body,
        compiler_params=pltpu.CompilerParams(needs_layout_passes=False),
        out_type=jax.ShapeDtypeStruct((B, D), jnp.float32),
        mesh=mesh,
        scratch_types=[
            pltpu.VMEM((MAXK,), jnp.int32),
            pltpu.VMEM((MAXK,), jnp.float32),
            pltpu.VMEM((MAXK * 16,), jnp.float32),
            pltpu.VMEM((MAXK, D), jnp.float32),
            pltpu.VMEM((D,), jnp.float32),
            pltpu.SemaphoreType.DMA,
        ],
    )
    return f(params_tbl, top_idx, weights)


def kernel(hidden, params, keys_tbl, W_router, b_router, k_dynamic, max_k):
    query = _pool_router(hidden, W_router, b_router)
    scores = _scores(query, keys_tbl)
    top_vals, top_idx = _topk_sc(scores)
    weights = _softmax(top_vals, k_dynamic, max_k)
    return _aggregate_sc(params, top_idx, weights)


# speedup vs baseline: 1.6393x; 1.6393x over previous
import functools
import math

import jax
import jax.numpy as jnp
from jax import lax
from jax.experimental import pallas as pl
from jax.experimental.pallas import tpu as pltpu
from jax.experimental.pallas import tpu_sc as plsc

B, T, D = 128, 256, 1024
V = 100000
VPAD = 100352
MAXK = 64

NW = 32
RPW = B // NW
CH = 12544
NCH = VPAD // CH
NGRP = CH // 128
RES = 320
NRV = RES // 16
TRIG = 192
NEG_INIT = -3.4e38


def _pool_router_body(hidden_ref, w_ref, b_ref, out_ref, acc_ref):
    t = pl.program_id(0)
    nt = pl.num_programs(0)

    @pl.when(t == 0)
    def _init():
        acc_ref[...] = jnp.zeros_like(acc_ref)

    acc_ref[...] += jnp.sum(hidden_ref[...], axis=1)

    @pl.when(t == nt - 1)
    def _fin():
        pooled = acc_ref[...] * (1.0 / T)
        out_ref[...] = (
            jnp.dot(pooled, w_ref[...], preferred_element_type=jnp.float32)
            + b_ref[...][None, :]
        )


def _pool_router(hidden, w, b):
    tb = 32
    return pl.pallas_call(
        _pool_router_body,
        grid=(T // tb,),
        in_specs=[
            pl.BlockSpec((B, tb, D), lambda t: (0, t, 0)),
            pl.BlockSpec((D, D), lambda t: (0, 0)),
            pl.BlockSpec((D,), lambda t: (0,)),
        ],
        out_specs=pl.BlockSpec((B, D), lambda t: (0, 0)),
        out_shape=jax.ShapeDtypeStruct((B, D), jnp.float32),
        scratch_shapes=[pltpu.VMEM((B, D), jnp.float32)],
    )(hidden, w, b)


def _scores_body(q_ref, k_ref, out_ref, *, vb):
    j = pl.program_id(0)
    s = lax.dot_general(
        q_ref[...], k_ref[...],
        dimension_numbers=(((1,), (1,)), ((), ())),
        preferred_element_type=jnp.float32,
    ) * (1.0 / math.sqrt(D))
    col = j * vb + lax.broadcasted_iota(jnp.int32, (B, vb), 1)
    out_ref[...] = jnp.where(col < V, s, -3.0e38)


def _scores(query, keys_tbl):
    vb = 2048
    return pl.pallas_call(
        functools.partial(_scores_body, vb=vb),
        grid=(VPAD // vb,),
        in_specs=[
            pl.BlockSpec((B, D), lambda j: (0, 0)),
            pl.BlockSpec((vb, D), lambda j: (j, 0)),
        ],
        out_specs=pl.BlockSpec((B, vb), lambda j: (0, j)),
        out_shape=jax.ShapeDtypeStruct((B, VPAD), jnp.float32),
    )(query, keys_tbl)


def _f2k(v):
    u = lax.bitcast_convert_type(v, jnp.uint32)
    return jnp.where(u >= jnp.uint32(0x80000000), ~u,
                     u | jnp.uint32(0x80000000))


def _iota16():
    return lax.iota(jnp.int32, 16)


def _rebuild(cand_v, cand_i, key_b, tie_v, tie_i, cnt):
    iot = _iota16()
    for j in range(NRV):
        vj = cand_v[pl.ds(j * 16, 16)]
        kj = _f2k(vj)
        lane = j * 16 + iot
        key_b[pl.ds(j * 16, 16)] = jnp.where(lane < cnt, kj, jnp.uint32(0))

    def bit_body(i, tk):
        cand_t = tk | (jnp.uint32(1) << (jnp.uint32(31) - i.astype(jnp.uint32)))
        tot = jnp.zeros((16,), jnp.int32)
        for j in range(NRV):
            kj = key_b[pl.ds(j * 16, 16)]
            tot = tot + (kj >= cand_t).astype(jnp.int32)
        return jnp.where(jnp.sum(tot) >= MAXK, cand_t, tk)

    tkey = lax.fori_loop(0, 32, bit_body, jnp.uint32(0))

    n_gt = jnp.int32(0)
    n_eq = jnp.int32(0)
    for j in range(NRV):
        kj = key_b[pl.ds(j * 16, 16)]
        vj = cand_v[pl.ds(j * 16, 16)]
        ij = cand_i[pl.ds(j * 16, 16)]
        mgt = kj > tkey
        meq = kj == tkey
        rgt = jnp.cumsum(mgt.astype(jnp.int32))
        req = jnp.cumsum(meq.astype(jnp.int32))
        pgt = n_gt + rgt - 1
        peq = n_eq + req - 1
        plsc.store_scatter(cand_v, [pgt], vj, mask=mgt)
        plsc.store_scatter(cand_i, [pgt], ij, mask=mgt)
        mke = meq & (peq < MAXK)
        plsc.store_scatter(tie_v, [peq], vj, mask=mke)
        plsc.store_scatter(tie_i, [peq], ij, mask=mke)
        n_gt = n_gt + rgt[15]
        n_eq = n_eq + req[15]

    need = MAXK - n_gt
    for jj in range(MAXK // 16):
        lidx = jj * 16 + iot
        mk = lidx < need
        tv = tie_v[pl.ds(jj * 16, 16)]
        ti = tie_i[pl.ds(jj * 16, 16)]
        plsc.store_scatter(cand_v, [n_gt + lidx], tv, mask=mk)
        plsc.store_scatter(cand_i, [n_gt + lidx], ti, mask=mk)

    tvv = jnp.full((16,), tkey, jnp.uint32)
    uv = jnp.where(tvv < jnp.uint32(0x80000000), ~tvv,
                   tvv ^ jnp.uint32(0x80000000))
    tf = lax.bitcast_convert_type(uv, jnp.float32)
    return tf, jnp.int32(MAXK)


def _topk_body(scores, out_v, out_i,
               buf0, buf1, cand_v, cand_i, key_b, tie_v, tie_i, key_s,
               ov, oi, sem0, sem1):
    wid = lax.axis_index("s") * 2 + lax.axis_index("c")
    iot = _iota16()

    def row_body(r, _carry):
        row = wid * RPW + r

        def src(c):
            return scores.at[row, pl.ds(c * CH, CH)]

        def process(bref, cbase, tv, cnt):
            def grp(g, carry2):
                tv, cnt = carry2
                tv, cnt = lax.cond(
                    cnt > TRIG,
                    lambda a, b: _rebuild(cand_v, cand_i, key_b, tie_v,
                                          tie_i, b),
                    lambda a, b: (a, b), tv, cnt)
                off = g * 128
                vs = [bref[pl.ds(off + 16 * j, 16)] for j in range(8)]
                m = vs[0]
                for j in range(1, 8):
                    m = jnp.maximum(m, vs[j])
                nhot = plsc.all_reduce_population_count(m > tv)

                def slow(cnt_in):
                    cc = cnt_in
                    for j in range(8):
                        vj = bref[pl.ds(off + 16 * j, 16)]
                        mj = vj > tv
                        cj = plsc.all_reduce_population_count(mj)

                        def app(ci, vj=vj, mj=mj, j=j):
                            rk = jnp.cumsum(mj.astype(jnp.int32))
                            pos = ci + rk - 1
                            gi = cbase + off + 16 * j + iot
                            plsc.store_scatter(cand_v, [pos], vj, mask=mj)
                            plsc.store_scatter(cand_i, [pos], gi, mask=mj)
                            return ci + rk[15]

                        cc = lax.cond(cj[0] > 0, app, lambda ci: ci, cc)
                    return cc

                cnt = lax.cond(nhot[0] > 0, slow, lambda ci: ci, cnt)
                return tv, cnt

            return lax.fori_loop(0, NGRP, grp, (tv, cnt))

        pltpu.async_copy(src(0), buf0, sem0)

        def pair_body(p, carry):
            tv, cnt = carry
            c0 = 2 * p
            pltpu.make_async_copy(src(c0), buf0, sem0).wait()
            pltpu.async_copy(src(c0 + 1), buf1, sem1)
            tv, cnt = process(buf0, c0 * CH, tv, cnt)
            pltpu.make_async_copy(src(c0 + 1), buf1, sem1).wait()

            @pl.when(p < NCH // 2 - 1)
            def _pref():
                pltpu.async_copy(src(c0 + 2), buf0, sem0)

            tv, cnt = process(buf1, (c0 + 1) * CH, tv, cnt)
            return tv, cnt

        tv, cnt = lax.fori_loop(
            0, NCH // 2, pair_body,
            (jnp.full((16,), jnp.float32(NEG_INIT)), jnp.int32(0)))

        tv, cnt = _rebuild(cand_v, cand_i, key_b, tie_v, tie_i, cnt)

        for j in range(MAXK // 16):
            kj = _f2k(cand_v[pl.ds(j * 16, 16)])
            key_s[pl.ds(j * 16, 16)] = lax.bitcast_convert_type(
                kj ^ jnp.uint32(0x80000000), jnp.int32)

        lane0 = iot == 0

        def sort_step(s, _c):
            ks = [key_s[pl.ds(16 * j, 16)] for j in range(4)]
            mx = jnp.maximum(jnp.maximum(ks[0], ks[1]),
                             jnp.maximum(ks[2], ks[3]))
            mxs = jnp.max(mx)
            posv = jnp.full((16,), jnp.int32(127))
            for j in range(4):
                posv = jnp.minimum(
                    posv, jnp.where(ks[j] == mxs, 16 * j + iot, 127))
            p = jnp.min(posv)
            pv = jnp.full((16,), p)
            val = plsc.load_gather(cand_v, [pv])
            idx = plsc.load_gather(cand_i, [pv])
            sv = jnp.full((16,), s)
            plsc.store_scatter(ov, [sv], val, mask=lane0)
            plsc.store_scatter(oi, [sv], idx, mask=lane0)
            plsc.store_scatter(key_s, [pv],
                               jnp.full((16,), jnp.int32(-2147483648)),
                               mask=lane0)
            return 0

        lax.fori_loop(0, MAXK, sort_step, 0)
        pltpu.sync_copy(ov, out_v.at[row])
        pltpu.sync_copy(oi, out_i.at[row])
        return 0

    lax.fori_loop(0, RPW, row_body, 0)


def _topk_sc(scores):
    mesh = plsc.VectorSubcoreMesh(core_axis_name="c", subcore_axis_name="s",
                                  num_cores=2, num_subcores=16)
    f = pl.kernel(
        _topk_body,
        compiler_params=pltpu.CompilerParams(needs_layout_passes=False),
        out_type=(jax.ShapeDtypeStruct((B, MAXK), jnp.float32),
                  jax.ShapeDtypeStruct((B, MAXK), jnp.int32)),
        mesh=mesh,
        scratch_types=[
            pltpu.VMEM((CH,), jnp.float32),
            pltpu.VMEM((CH,), jnp.float32),
            pltpu.VMEM((RES,), jnp.float32),
            pltpu.VMEM((RES,), jnp.int32),
            pltpu.VMEM((RES,), jnp.uint32),
            pltpu.VMEM((MAXK,), jnp.float32),
            pltpu.VMEM((MAXK,), jnp.int32),
            pltpu.VMEM((MAXK,), jnp.int32),
            pltpu.VMEM((MAXK,), jnp.float32),
            pltpu.VMEM((MAXK,), jnp.int32),
            pltpu.SemaphoreType.DMA,
            pltpu.SemaphoreType.DMA,
        ],
    )
    return f(scores)


def _softmax_body(v_ref, kd_ref, out_ref):
    kd = jnp.minimum(kd_ref[0], kd_ref[1])
    iota = lax.broadcasted_iota(jnp.int32, (B, MAXK), 1)
    vals = jnp.where(iota < kd, v_ref[...], -jnp.inf)
    m = jnp.max(vals, axis=1, keepdims=True)
    e = jnp.where(iota < kd, jnp.exp(vals - m), 0.0)
    out_ref[...] = e / jnp.sum(e, axis=1, keepdims=True)


def _softmax(top_vals, k_dynamic, max_k):
    kd = jnp.asarray([k_dynamic, max_k], jnp.int32).reshape(2)
    return pl.pallas_call(
        _softmax_body,
        grid=(1,),
        in_specs=[
            pl.BlockSpec((B, MAXK), lambda i: (0, 0)),
            pl.BlockSpec(memory_space=pltpu.SMEM),
        ],
        out_specs=pl.BlockSpec((B, MAXK), lambda i: (0, 0)),
        out_shape=jax.ShapeDtypeStruct((B, MAXK), jnp.float32),
    )(top_vals, kd)


def _lane_splat(x, j):
    dn = lax.GatherDimensionNumbers(offset_dims=(), collapsed_slice_dims=(0,),
                                    start_index_map=(0,))
    idx = jnp.full((16,), j, jnp.int32)
    return lax.gather(x, idx[:, None], dn, (1,),
                      mode=lax.GatherScatterMode.PROMISE_IN_BOUNDS)


def _agg_body(params, top_idx, weights, out,
              idx_v, w_v, wb, rows_v, acc, sem):
    wid = lax.axis_index("s") * 2 + lax.axis_index("c")

    def row_body(r, _carry):
        row = wid * RPW + r
        pltpu.sync_copy(top_idx.at[row], idx_v)
        pltpu.sync_copy(weights.at[row], w_v)
        cp = pltpu.async_copy(params.at[idx_v], rows_v, sem)
        for kk in range(MAXK // 16):
            wv = w_v[pl.ds(kk * 16, 16)]
            for j in range(16):
                wb[pl.ds((kk * 16 + j) * 16, 16)] = _lane_splat(wv, j)
        cp.wait()

        def d_body(dd, _c):
            a = jnp.zeros((16,), jnp.float32)
            for k in range(MAXK):
                a = a + wb[pl.ds(k * 16, 16)] * rows_v[k, pl.ds(dd * 16, 16)]
            acc[pl.ds(dd * 16, 16)] = a
            return 0

        lax.fori_loop(0, D // 16, d_body, 0)
        pltpu.sync_copy(acc, out.at[row])
        return 0

    lax.fori_loop(0, RPW, row_body, 0)


def _aggregate_sc(params_tbl, top_idx, weights):
    mesh = plsc.VectorSubcoreMesh(core_axis_name="c", subcore_axis_name="s",
                                  num_cores=2, num_subcores=16)
    f = pl.kernel(
        _agg_body,
        compiler_params=pltpu.CompilerParams(needs_layout_passes=False),
        out_type=jax.ShapeDtypeStruct((B, D), jnp.float32),
        mesh=mesh,
        scratch_types=[
            pltpu.VMEM((MAXK,), jnp.int32),
            pltpu.VMEM((MAXK,), jnp.float32),
            pltpu.VMEM((MAXK * 16,), jnp.float32),
            pltpu.VMEM((MAXK, D), jnp.float32),
            pltpu.VMEM((D,), jnp.float32),
            pltpu.SemaphoreType.DMA,
        ],
    )
    return f(params_tbl, top_idx, weights)


def kernel(hidden, params, keys_tbl, W_router, b_router, k_dynamic, max_k):
    query = _pool_router(hidden, W_router, b_router)
    scores = _scores(query, keys_tbl)
    top_vals, top_idx = _topk_sc(scores)
    weights = _softmax(top_vals, k_dynamic, max_k)
    return _aggregate_sc(params, top_idx, weights)


# speedup vs baseline: 2.6364x; 1.6083x over previous
import functools
import math

import jax
import jax.numpy as jnp
from jax import lax
from jax.experimental import pallas as pl
from jax.experimental.pallas import tpu as pltpu
from jax.experimental.pallas import tpu_sc as plsc

B, T, D = 128, 256, 1024
V = 100000
VPAD = 100352
MAXK = 64

NW = 32
RPW = B // NW
CH = 12544
NCH = VPAD // CH
NGRP = CH // 128
RES = 320
NRV = RES // 16
TRIG = 192
NEG_INIT = -3.4e38


def _pool_body(hidden_ref, out_ref):
    out_ref[...] = jnp.mean(hidden_ref[...], axis=1)


def _router_body(p_ref, w_ref, b_ref, out_ref):
    out_ref[...] = (
        jnp.dot(p_ref[...], w_ref[...], preferred_element_type=jnp.float32)
        + b_ref[...][None, :]
    )


def _pool_router(hidden, w, b):
    pooled = pl.pallas_call(
        _pool_body,
        grid=(8,),
        in_specs=[pl.BlockSpec((B, T, D // 8), lambda j: (0, 0, j))],
        out_specs=pl.BlockSpec((B, D // 8), lambda j: (0, j)),
        out_shape=jax.ShapeDtypeStruct((B, D), jnp.float32),
    )(hidden)
    return pl.pallas_call(
        _router_body,
        grid=(1,),
        in_specs=[
            pl.BlockSpec((B, D), lambda j: (0, 0)),
            pl.BlockSpec((D, D), lambda j: (0, 0)),
            pl.BlockSpec((D,), lambda j: (0,)),
        ],
        out_specs=pl.BlockSpec((B, D), lambda j: (0, 0)),
        out_shape=jax.ShapeDtypeStruct((B, D), jnp.float32),
    )(pooled, w, b)


def _scores_body(q_ref, k_ref, out_ref, gm_ref, *, vb):
    j = pl.program_id(0)
    s = lax.dot_general(
        q_ref[...], k_ref[...],
        dimension_numbers=(((1,), (1,)), ((), ())),
        preferred_element_type=jnp.float32,
    ) * (1.0 / math.sqrt(D))
    col = j * vb + lax.broadcasted_iota(jnp.int32, (B, vb), 1)
    sm = jnp.where(col < V, s, -3.0e38)
    out_ref[...] = sm
    gm_ref[0] = jnp.max(sm.reshape(B, vb // 128, 128), axis=2)


def _scores(query, keys_tbl):
    vb = 2048
    nsteps = VPAD // vb
    return pl.pallas_call(
        functools.partial(_scores_body, vb=vb),
        grid=(nsteps,),
        in_specs=[
            pl.BlockSpec((B, D), lambda j: (0, 0)),
            pl.BlockSpec((vb, D), lambda j: (j, 0)),
        ],
        out_specs=(
            pl.BlockSpec((B, vb), lambda j: (0, j)),
            pl.BlockSpec((1, B, vb // 128), lambda j: (j, 0, 0)),
        ),
        out_shape=(
            jax.ShapeDtypeStruct((B, VPAD), jnp.float32),
            jax.ShapeDtypeStruct((nsteps, B, vb // 128), jnp.float32),
        ),
    )(query, keys_tbl)


def _f2k(v):
    u = lax.bitcast_convert_type(v, jnp.uint32)
    return jnp.where(u >= jnp.uint32(0x80000000), ~u,
                     u | jnp.uint32(0x80000000))


def _iota16():
    return lax.iota(jnp.int32, 16)


def _rebuild(cand_v, cand_i, key_b, tie_v, tie_i, cnt):
    iot = _iota16()
    for j in range(NRV):
        vj = cand_v[pl.ds(j * 16, 16)]
        kj = _f2k(vj)
        lane = j * 16 + iot
        key_b[pl.ds(j * 16, 16)] = jnp.where(lane < cnt, kj, jnp.uint32(0))

    def bit_body(i, tk):
        cand_t = tk | (jnp.uint32(1) << (jnp.uint32(31) - i.astype(jnp.uint32)))
        tot = jnp.zeros((16,), jnp.int32)
        for j in range(NRV):
            kj = key_b[pl.ds(j * 16, 16)]
            tot = tot + (kj >= cand_t).astype(jnp.int32)
        return jnp.where(jnp.sum(tot) >= MAXK, cand_t, tk)

    tkey = lax.fori_loop(0, 32, bit_body, jnp.uint32(0))

    n_gt = jnp.int32(0)
    n_eq = jnp.int32(0)
    for j in range(NRV):
        kj = key_b[pl.ds(j * 16, 16)]
        vj = cand_v[pl.ds(j * 16, 16)]
        ij = cand_i[pl.ds(j * 16, 16)]
        mgt = kj > tkey
        meq = kj == tkey
        rgt = jnp.cumsum(mgt.astype(jnp.int32))
        req = jnp.cumsum(meq.astype(jnp.int32))
        pgt = n_gt + rgt - 1
        peq = n_eq + req - 1
        plsc.store_scatter(cand_v, [pgt], vj, mask=mgt)
        plsc.store_scatter(cand_i, [pgt], ij, mask=mgt)
        mke = meq & (peq < MAXK)
        plsc.store_scatter(tie_v, [peq], vj, mask=mke)
        plsc.store_scatter(tie_i, [peq], ij, mask=mke)
        n_gt = n_gt + rgt[15]
        n_eq = n_eq + req[15]

    need = MAXK - n_gt
    for jj in range(MAXK // 16):
        lidx = jj * 16 + iot
        mk = lidx < need
        tv = tie_v[pl.ds(jj * 16, 16)]
        ti = tie_i[pl.ds(jj * 16, 16)]
        plsc.store_scatter(cand_v, [n_gt + lidx], tv, mask=mk)
        plsc.store_scatter(cand_i, [n_gt + lidx], ti, mask=mk)

    tvv = jnp.full((16,), tkey, jnp.uint32)
    uv = jnp.where(tvv < jnp.uint32(0x80000000), ~tvv,
                   tvv ^ jnp.uint32(0x80000000))
    tf = lax.bitcast_convert_type(uv, jnp.float32)
    return tf, jnp.int32(MAXK)


NG = VPAD // 128
NGV = NG // 16
WLSZ = NG + 64


def _topk_body(scores2d, gmax, out_v, out_i,
               gm_v, gk, wl, idx64, batch, cand_v, cand_i, key_b,
               tie_v, tie_i, key_s, ov, oi, sem):
    wid = lax.axis_index("s") * 2 + lax.axis_index("c")
    iot = _iota16()
    lane0 = iot == 0

    def row_body(r, _carry):
        row = wid * RPW + r
        rbase = row * NG
        pltpu.sync_copy(gmax.at[:, row, :], gm_v)

        for j in range(NGV):
            gk[pl.ds(j * 16, 16)] = _f2k(gm_v[j])

        def bit_body(i, tk):
            cand_t = tk | (jnp.uint32(1)
                           << (jnp.uint32(31) - i.astype(jnp.uint32)))
            tot = jnp.zeros((16,), jnp.int32)
            for j in range(NGV):
                tot = tot + (gk[pl.ds(j * 16, 16)] >= cand_t).astype(
                    jnp.int32)
            return jnp.where(jnp.sum(tot) >= MAXK, cand_t, tk)

        tk0 = lax.fori_loop(0, 32, bit_body, jnp.uint32(0))

        tm1 = jnp.full((16,), tk0 - jnp.uint32(1))
        uv = jnp.where(tm1 < jnp.uint32(0x80000000), ~tm1,
                       tm1 ^ jnp.uint32(0x80000000))
        tv0 = lax.bitcast_convert_type(uv, jnp.float32)

        base = jnp.int32(0)
        for j in range(NGV):
            hot = gk[pl.ds(j * 16, 16)] >= tk0
            rk = jnp.cumsum(hot.astype(jnp.int32))
            pos = base + rk - 1
            plsc.store_scatter(wl, [pos], rbase + j * 16 + iot, mask=hot)
            base = base + rk[15]
        nhot = base
        for k in range(4):
            plsc.store_scatter(wl, [nhot + k * 16 + iot],
                               jnp.full((16,), rbase))

        def batch_body(b, carry):
            tv, cnt = carry
            for k in range(4):
                idx64[pl.ds(k * 16, 16)] = wl[pl.ds(b * 64 + k * 16, 16)]
            pltpu.async_copy(scores2d.at[idx64], batch, sem).wait()

            def slot(ss, c2):
                tv, cnt = c2
                tv, cnt = lax.cond(
                    cnt > TRIG,
                    lambda a, b2: _rebuild(cand_v, cand_i, key_b, tie_v,
                                           tie_i, b2),
                    lambda a, b2: (a, b2), tv, cnt)
                ws = b * 64 + ss

                def proc(tv_i, cnt_i):
                    gidv = plsc.load_gather(wl, [jnp.full((16,), ws)])
                    gloc = (gidv - rbase) * 128
                    cc = cnt_i
                    for jj in range(8):
                        vj = batch[ss, pl.ds(jj * 16, 16)]
                        mj = vj > tv_i
                        cj = plsc.all_reduce_population_count(mj)

                        def app(ci, vj=vj, mj=mj, jj=jj):
                            rk = jnp.cumsum(mj.astype(jnp.int32))
                            pos = ci + rk - 1
                            gi = gloc + jj * 16 + iot
                            plsc.store_scatter(cand_v, [pos], vj, mask=mj)
                            plsc.store_scatter(cand_i, [pos], gi, mask=mj)
                            return ci + rk[15]

                        cc = lax.cond(cj[0] > 0, app, lambda ci: ci, cc)
                    return tv_i, cc

                return lax.cond(ws < nhot, proc,
                                lambda a, b2: (a, b2), tv, cnt)

            return lax.fori_loop(0, 64, slot, (tv, cnt))

        nb = (nhot + 63) // 64
        tv, cnt = lax.fori_loop(0, nb, batch_body, (tv0, jnp.int32(0)))

        tv, cnt = _rebuild(cand_v, cand_i, key_b, tie_v, tie_i, cnt)

        for j in range(MAXK // 16):
            kj = _f2k(cand_v[pl.ds(j * 16, 16)])
            key_s[pl.ds(j * 16, 16)] = lax.bitcast_convert_type(
                kj ^ jnp.uint32(0x80000000), jnp.int32)

        def sort_step(s2, _c):
            ks = [key_s[pl.ds(16 * j, 16)] for j in range(4)]
            mx = jnp.maximum(jnp.maximum(ks[0], ks[1]),
                             jnp.maximum(ks[2], ks[3]))
            mxs = jnp.max(mx)
            posv = jnp.full((16,), jnp.int32(127))
            for j in range(4):
                posv = jnp.minimum(
                    posv, jnp.where(ks[j] == mxs, 16 * j + iot, 127))
            p = jnp.min(posv)
            pv = jnp.full((16,), p)
            val = plsc.load_gather(cand_v, [pv])
            idx = plsc.load_gather(cand_i, [pv])
            sv = jnp.full((16,), s2)
            plsc.store_scatter(ov, [sv], val, mask=lane0)
            plsc.store_scatter(oi, [sv], idx, mask=lane0)
            plsc.store_scatter(key_s, [pv],
                               jnp.full((16,), jnp.int32(-2147483648)),
                               mask=lane0)
            return 0

        lax.fori_loop(0, MAXK, sort_step, 0)
        pltpu.sync_copy(ov, out_v.at[row])
        pltpu.sync_copy(oi, out_i.at[row])
        return 0

    lax.fori_loop(0, RPW, row_body, 0)


def _topk_sc(scores, gmax):
    scores2d = scores.reshape(B * NG, 128)
    mesh = plsc.VectorSubcoreMesh(core_axis_name="c", subcore_axis_name="s",
                                  num_cores=2, num_subcores=16)
    f = pl.kernel(
        _topk_body,
        compiler_params=pltpu.CompilerParams(needs_layout_passes=False),
        out_type=(jax.ShapeDtypeStruct((B, MAXK), jnp.float32),
                  jax.ShapeDtypeStruct((B, MAXK), jnp.int32)),
        mesh=mesh,
        scratch_types=[
            pltpu.VMEM((NGV, 16), jnp.float32),
            pltpu.VMEM((NG,), jnp.uint32),
            pltpu.VMEM((WLSZ,), jnp.int32),
            pltpu.VMEM((64,), jnp.int32),
            pltpu.VMEM((64, 128), jnp.float32),
            pltpu.VMEM((RES,), jnp.float32),
            pltpu.VMEM((RES,), jnp.int32),
            pltpu.VMEM((RES,), jnp.uint32),
            pltpu.VMEM((MAXK,), jnp.float32),
            pltpu.VMEM((MAXK,), jnp.int32),
            pltpu.VMEM((MAXK,), jnp.int32),
            pltpu.VMEM((MAXK,), jnp.float32),
            pltpu.VMEM((MAXK,), jnp.int32),
            pltpu.SemaphoreType.DMA,
        ],
    )
    return f(scores2d, gmax)


def _softmax_body(v_ref, kd_ref, out_ref):
    kd = jnp.minimum(kd_ref[0], kd_ref[1])
    iota = lax.broadcasted_iota(jnp.int32, (B, MAXK), 1)
    vals = jnp.where(iota < kd, v_ref[...], -jnp.inf)
    m = jnp.max(vals, axis=1, keepdims=True)
    e = jnp.where(iota < kd, jnp.exp(vals - m), 0.0)
    out_ref[...] = e / jnp.sum(e, axis=1, keepdims=True)


def _softmax(top_vals, k_dynamic, max_k):
    kd = jnp.asarray([k_dynamic, max_k], jnp.int32).reshape(2)
    return pl.pallas_call(
        _softmax_body,
        grid=(1,),
        in_specs=[
            pl.BlockSpec((B, MAXK), lambda i: (0, 0)),
            pl.BlockSpec(memory_space=pltpu.SMEM),
        ],
        out_specs=pl.BlockSpec((B, MAXK), lambda i: (0, 0)),
        out_shape=jax.ShapeDtypeStruct((B, MAXK), jnp.float32),
    )(top_vals, kd)


def _lane_splat(x, j):
    dn = lax.GatherDimensionNumbers(offset_dims=(), collapsed_slice_dims=(0,),
                                    start_index_map=(0,))
    idx = jnp.full((16,), j, jnp.int32)
    return lax.gather(x, idx[:, None], dn, (1,),
                      mode=lax.GatherScatterMode.PROMISE_IN_BOUNDS)


def _agg_body(params, top_idx, weights, out,
              idx_v, w_v, wb, rows_v, acc, sem):
    wid = lax.axis_index("s") * 2 + lax.axis_index("c")

    def row_body(r, _carry):
        row = wid * RPW + r
        pltpu.sync_copy(top_idx.at[row], idx_v)
        pltpu.sync_copy(weights.at[row], w_v)
        cp = pltpu.async_copy(params.at[idx_v], rows_v, sem)
        for kk in range(MAXK // 16):
            wv = w_v[pl.ds(kk * 16, 16)]
            for j in range(16):
                wb[pl.ds((kk * 16 + j) * 16, 16)] = _lane_splat(wv, j)
        cp.wait()

        def d_body(dd, _c):
            a = jnp.zeros((16,), jnp.float32)
            for k in range(MAXK):
                a = a + wb[pl.ds(k * 16, 16)] * rows_v[k, pl.ds(dd * 16, 16)]
            acc[pl.ds(dd * 16, 16)] = a
            return 0

        lax.fori_loop(0, D // 16, d_body, 0)
        pltpu.sync_copy(acc, out.at[row])
        return 0

    lax.fori_loop(0, RPW, row_body, 0)


def _aggregate_sc(params_tbl, top_idx, weights):
    mesh = plsc.VectorSubcoreMesh(core_axis_name="c", subcore_axis_name="s",
                                  num_cores=2, num_subcores=16)
    f = pl.kernel(
        _agg_body,
        compiler_params=pltpu.CompilerParams(needs_layout_passes=False),
        out_type=jax.ShapeDtypeStruct((B, D), jnp.float32),
        mesh=mesh,
        scratch_types=[
            pltpu.VMEM((MAXK,), jnp.int32),
            pltpu.VMEM((MAXK,), jnp.float32),
            pltpu.VMEM((MAXK * 16,), jnp.float32),
            pltpu.VMEM((MAXK, D), jnp.float32),
            pltpu.VMEM((D,), jnp.float32),
            pltpu.SemaphoreType.DMA,
        ],
    )
    return f(params_tbl, top_idx, weights)


def kernel(hidden, params, keys_tbl, W_router, b_router, k_dynamic, max_k):
    query = _pool_router(hidden, W_router, b_router)
    scores, gmax = _scores(query, keys_tbl)
    top_vals, top_idx = _topk_sc(scores, gmax)
    weights = _softmax(top_vals, k_dynamic, max_k)
    return _aggregate_sc(params, top_idx, weights)
